# speedup vs baseline: 1.3904x; 1.3904x over previous
import jax
import jax.numpy as jnp
from jax.experimental import pallas as pl
from jax.experimental.pallas import tpu as pltpu

N = 2048
D_MODEL = 1024
D_FF = 2048
E = 8
TOP_K = 2


def _moe_body(ei_ref, ew_ref, x_ref, w1_ref, w2_ref, o_ref):
    e = pl.program_id(0)

    @pl.when(e == 0)
    def _init():
        o_ref[...] = jnp.zeros_like(o_ref)

    mask = ei_ref[...] == e
    w_col = jnp.sum(jnp.where(mask, ew_ref[...], 0.0), axis=1)

    x = x_ref[...]
    h = jax.nn.relu(
        jax.lax.dot_general(x, w1_ref[0], (((1,), (0,)), ((), ())),
                            preferred_element_type=jnp.float32)
    ).astype(jnp.bfloat16)
    y = jax.lax.dot_general(h, w2_ref[0], (((1,), (0,)), ((), ())),
                            preferred_element_type=jnp.float32)
    o_ref[...] += w_col[:, None] * y


def kernel(x, expert_weights, expert_indices, batch_size_per_expert, W1, W2):
    del batch_size_per_expert
    xb = x.astype(jnp.bfloat16)
    w1b = W1.astype(jnp.bfloat16)
    w2b = W2.astype(jnp.bfloat16)
    ei = expert_indices.astype(jnp.int32)

    out = pl.pallas_call(
        _moe_body,
        grid=(E,),
        in_specs=[
            pl.BlockSpec((N, TOP_K), lambda e: (0, 0)),
            pl.BlockSpec((N, TOP_K), lambda e: (0, 0)),
            pl.BlockSpec((N, D_MODEL), lambda e: (0, 0)),
            pl.BlockSpec((1, D_MODEL, D_FF), lambda e: (e, 0, 0)),
            pl.BlockSpec((1, D_FF, D_MODEL), lambda e: (e, 0, 0)),
        ],
        out_specs=pl.BlockSpec((N, D_MODEL), lambda e: (0, 0)),
        out_shape=jax.ShapeDtypeStruct((N, D_MODEL), jnp.float32),
    )(ei, expert_weights, xb, w1b, w2b)
    return out
